# baseline (speedup 1.0000x reference)
import functools

import jax
import jax.numpy as jnp
from jax import lax
from jax.experimental import pallas as pl
from jax.experimental.pallas import tpu as pltpu
from jax.experimental.pallas import tpu_sc as plsc

H, B, V, D = 200, 4096, 1000000, 32
NW = 32
UNITS = (H // 8) * (B // 128)
UNITS_W = UNITS // NW


def _rsqrt_newton(ss):
    xhalf = 0.5 * ss
    i = lax.bitcast_convert_type(ss, jnp.int32)
    i = jnp.int32(0x5F3759DF) - (i >> 1)
    y = lax.bitcast_convert_type(i, jnp.float32)
    y = y * (1.5 - xhalf * y * y)
    y = y * (1.5 - xhalf * y * y)
    y = y * (1.5 - xhalf * y * y)
    return y


def _make_sc_call():
    mesh_sc = plsc.VectorSubcoreMesh(core_axis_name="c", subcore_axis_name="s")

    @functools.partial(
        pl.kernel,
        out_type=jax.ShapeDtypeStruct((H, 4, B // 128, 8, 128), jnp.float32),
        mesh=mesh_sc,
        scratch_types=[
            pltpu.VMEM((8, 128), jnp.int32),
            pltpu.VMEM((1024, 32), jnp.float32),
            pltpu.VMEM((32, 128), jnp.float32),
            pltpu.SemaphoreType.DMA,
            pltpu.SemaphoreType.DMA,
        ],
        compiler_params=pltpu.CompilerParams(
            needs_layout_passes=False, use_tc_tiling_on_sc=False
        ),
    )
    def sck(w_hbm, a2_hbm, out_hbm, idx_v, grows_v, outb_v, gsem, osem):
        wid = lax.axis_index("s") * 2 + lax.axis_index("c")
        iota = lax.iota(jnp.int32, 16)

        def do_unit(u, carry):
            unit = wid * UNITS_W + u
            h8 = unit // (B // 128)
            b1 = unit % (B // 128)
            pltpu.sync_copy(
                a2_hbm.at[pl.ds(pl.multiple_of(unit * 8, 8), 8)], idx_v
            )
            copies = [
                pltpu.async_copy(
                    w_hbm.at[idx_v.at[hl]],
                    grows_v.at[pl.ds(hl * 128, 128), :],
                    gsem,
                )
                for hl in range(8)
            ]
            for cp in copies:
                cp.wait()

            for hl in range(8):
                def do_block(j, inner):
                    row_idx = hl * 128 + j * 16 + iota
                    colb = j * 16 + iota
                    c = iota
                    ss = jnp.zeros((16,), jnp.float32)
                    for _s in range(32):
                        g = plsc.load_gather(grows_v, [row_idx, c])
                        ss = ss + g * g
                        c = (c + 1) & 31
                    y = _rsqrt_newton(jnp.maximum(ss, 1e-24))
                    c = iota
                    for _s in range(32):
                        g = plsc.load_gather(grows_v, [row_idx, c])
                        plsc.store_scatter(outb_v, [c, colb], g * y)
                        c = (c + 1) & 31
                    return inner

                lax.fori_loop(0, 8, do_block, 0)

                h = h8 * 8 + hl
                ocopies = [
                    pltpu.async_copy(
                        outb_v.at[pl.ds(g * 8, 8), :],
                        out_hbm.at[h, g, b1],
                        osem,
                    )
                    for g in range(4)
                ]
                for cp in ocopies:
                    cp.wait()
            return carry

        lax.fori_loop(0, UNITS_W, do_unit, 0)

    return sck


def kernel(input, W):
    a2 = (
        jnp.transpose(input, (1, 0))
        .astype(jnp.int32)
        .reshape(25, 8, 32, 128)
        .transpose(0, 2, 1, 3)
        .reshape(6400, 128)
    )
    o5 = _make_sc_call()(W, a2)
    return jnp.transpose(o5, (0, 2, 4, 1, 3)).reshape(H, B, D)


# speedup vs baseline: 1.0851x; 1.0851x over previous
import functools

import jax
import jax.numpy as jnp
from jax import lax
from jax.experimental import pallas as pl
from jax.experimental.pallas import tpu as pltpu
from jax.experimental.pallas import tpu_sc as plsc

H, B, V, D = 200, 4096, 1000000, 32
NW = 32
UNITS = (H // 8) * (B // 128)
UNITS_W = UNITS // NW


def _rsqrt_newton(ss):
    xhalf = 0.5 * ss
    i = lax.bitcast_convert_type(ss, jnp.int32)
    i = jnp.int32(0x5F3759DF) - (i >> 1)
    y = lax.bitcast_convert_type(i, jnp.float32)
    y = y * (1.5 - xhalf * y * y)
    y = y * (1.5 - xhalf * y * y)
    y = y * (1.5 - xhalf * y * y)
    return y


def _make_sc_call():
    mesh_sc = plsc.VectorSubcoreMesh(core_axis_name="c", subcore_axis_name="s")

    @functools.partial(
        pl.kernel,
        out_type=jax.ShapeDtypeStruct((H, 4, B // 128, 8, 128), jnp.float32),
        mesh=mesh_sc,
        scratch_types=[
            pltpu.VMEM((16, 128), jnp.int32),
            pltpu.VMEM((2048, 32), jnp.float32),
            pltpu.VMEM((64, 128), jnp.float32),
            pltpu.SemaphoreType.DMA,
            pltpu.SemaphoreType.DMA,
            pltpu.SemaphoreType.DMA,
            pltpu.SemaphoreType.DMA,
        ],
        compiler_params=pltpu.CompilerParams(
            needs_layout_passes=False, use_tc_tiling_on_sc=False
        ),
    )
    def sck(
        w_hbm, a2_hbm, out_hbm, idx_v, grows_v, outb_v, gsem, isem, osem0, osem1
    ):
        osems = (osem0, osem1)
        wid = lax.axis_index("s") * 2 + lax.axis_index("c")
        iota = lax.iota(jnp.int32, 16)
        u0 = wid * UNITS_W

        def stage_idx(u, par):
            return pltpu.async_copy(
                a2_hbm.at[pl.ds(pl.multiple_of((u0 + u) * 8, 8), 8)],
                idx_v.at[pl.ds(pl.multiple_of(par * 8, 8), 8)],
                isem,
            )

        def drain_idx(par):
            pltpu.make_async_copy(
                a2_hbm.at[pl.ds(0, 8)],
                idx_v.at[pl.ds(pl.multiple_of(par * 8, 8), 8)],
                isem,
            ).wait()

        def fire_gathers(par):
            for hl in range(8):
                pltpu.async_copy(
                    w_hbm.at[idx_v.at[par * 8 + hl]],
                    grows_v.at[
                        pl.ds(pl.multiple_of(par * 1024 + hl * 128, 128), 128),
                        :,
                    ],
                    gsem,
                )

        def drain_gathers(par):
            for hl in range(8):
                pltpu.make_async_copy(
                    w_hbm.at[idx_v.at[par * 8 + hl]],
                    grows_v.at[
                        pl.ds(pl.multiple_of(par * 1024 + hl * 128, 128), 128),
                        :,
                    ],
                    gsem,
                ).wait()

        def drain_out(h_par):
            for g in range(4):
                pltpu.make_async_copy(
                    outb_v.at[pl.ds(h_par * 32 + g * 8, 8), :],
                    out_hbm.at[0, g, 0],
                    osems[h_par],
                ).wait()

        stage_idx(0, 0).wait()
        fire_gathers(0)
        stage_idx(1, 1)

        def do_unit(u, carry):
            unit = u0 + u
            h8 = unit // (B // 128)
            b1 = unit % (B // 128)
            pg = u & 1

            drain_gathers(pg)

            @pl.when(u < UNITS_W - 1)
            def _():
                drain_idx(1 - pg)
                fire_gathers(1 - pg)

            @pl.when(u < UNITS_W - 2)
            def _():
                stage_idx(u + 2, pg)

            for hl in range(8):
                hp = hl & 1
                if hl >= 2:
                    drain_out(hp)
                else:
                    @pl.when(u > 0)
                    def _():
                        drain_out(hp)

                def do_block(j, inner):
                    row_idx = pg * 1024 + hl * 128 + j * 16 + iota
                    colb = j * 16 + iota
                    c = iota
                    ss = jnp.zeros((16,), jnp.float32)
                    for _s in range(32):
                        g = plsc.load_gather(grows_v, [row_idx, c])
                        ss = ss + g * g
                        c = (c + 1) & 31
                    y = _rsqrt_newton(jnp.maximum(ss, 1e-24))
                    c = iota
                    for _s in range(32):
                        g = plsc.load_gather(grows_v, [row_idx, c])
                        plsc.store_scatter(outb_v, [hp * 32 + c, colb], g * y)
                        c = (c + 1) & 31
                    return inner

                lax.fori_loop(0, 8, do_block, 0)

                h = h8 * 8 + hl
                for g in range(4):
                    pltpu.async_copy(
                        outb_v.at[pl.ds(hp * 32 + g * 8, 8), :],
                        out_hbm.at[h, g, b1],
                        osems[hp],
                    )
            return carry

        lax.fori_loop(0, UNITS_W, do_unit, 0)
        drain_out(0)
        drain_out(1)

    return sck


def kernel(input, W):
    a2 = (
        jnp.transpose(input, (1, 0))
        .astype(jnp.int32)
        .reshape(25, 8, 32, 128)
        .transpose(0, 2, 1, 3)
        .reshape(6400, 128)
    )
    o5 = _make_sc_call()(W, a2)
    return jnp.transpose(o5, (0, 2, 4, 1, 3)).reshape(H, B, D)


# speedup vs baseline: 1.1765x; 1.0842x over previous
import functools

import jax
import jax.numpy as jnp
from jax import lax
from jax.experimental import pallas as pl
from jax.experimental.pallas import tpu as pltpu
from jax.experimental.pallas import tpu_sc as plsc

H, B, V, D = 200, 4096, 1000000, 32
NW = 32
UNITS = (H // 8) * (B // 128)
UNITS_W = UNITS // NW


def _rsqrt_newton(ss):
    xhalf = 0.5 * ss
    i = lax.bitcast_convert_type(ss, jnp.int32)
    i = jnp.int32(0x5F3759DF) - (i >> 1)
    y = lax.bitcast_convert_type(i, jnp.float32)
    y = y * (1.5 - xhalf * y * y)
    y = y * (1.5 - xhalf * y * y)
    y = y * (1.5 - xhalf * y * y)
    return y


def _make_sc_call():
    mesh_sc = plsc.VectorSubcoreMesh(core_axis_name="c", subcore_axis_name="s")

    @functools.partial(
        pl.kernel,
        out_type=jax.ShapeDtypeStruct((H, 4, B // 128, 8, 128), jnp.float32),
        mesh=mesh_sc,
        scratch_types=[
            pltpu.VMEM((16, 128), jnp.int32),
            pltpu.VMEM((2048, 32), jnp.float32),
            pltpu.VMEM((64, 128), jnp.float32),
            pltpu.SemaphoreType.DMA,
            pltpu.SemaphoreType.DMA,
            pltpu.SemaphoreType.DMA,
            pltpu.SemaphoreType.DMA,
        ],
        compiler_params=pltpu.CompilerParams(
            needs_layout_passes=False, use_tc_tiling_on_sc=False
        ),
    )
    def sck(
        w_hbm, a2_hbm, out_hbm, idx_v, grows_v, outb_v, gsem, isem, osem0, osem1
    ):
        osems = (osem0, osem1)
        wid = lax.axis_index("s") * 2 + lax.axis_index("c")
        iota = lax.iota(jnp.int32, 16)
        u0 = wid * UNITS_W

        def stage_idx(u, par):
            return pltpu.async_copy(
                a2_hbm.at[pl.ds(pl.multiple_of((u0 + u) * 8, 8), 8)],
                idx_v.at[pl.ds(pl.multiple_of(par * 8, 8), 8)],
                isem,
            )

        def drain_idx(par):
            pltpu.make_async_copy(
                a2_hbm.at[pl.ds(0, 8)],
                idx_v.at[pl.ds(pl.multiple_of(par * 8, 8), 8)],
                isem,
            ).wait()

        def fire_gathers(par):
            for hl in range(8):
                pltpu.async_copy(
                    w_hbm.at[idx_v.at[par * 8 + hl]],
                    grows_v.at[
                        pl.ds(pl.multiple_of(par * 1024 + hl * 128, 128), 128),
                        :,
                    ],
                    gsem,
                )

        def drain_gathers(par):
            for hl in range(8):
                pltpu.make_async_copy(
                    w_hbm.at[idx_v.at[par * 8 + hl]],
                    grows_v.at[
                        pl.ds(pl.multiple_of(par * 1024 + hl * 128, 128), 128),
                        :,
                    ],
                    gsem,
                ).wait()

        def drain_out(h_par):
            for g in range(4):
                pltpu.make_async_copy(
                    outb_v.at[pl.ds(h_par * 32 + g * 8, 8), :],
                    out_hbm.at[0, g, 0],
                    osems[h_par],
                ).wait()

        stage_idx(0, 0).wait()
        fire_gathers(0)
        stage_idx(1, 1)

        def do_unit(u, carry):
            unit = u0 + u
            h8 = unit // (B // 128)
            b1 = unit % (B // 128)
            pg = u & 1

            drain_gathers(pg)

            @pl.when(u < UNITS_W - 1)
            def _():
                drain_idx(1 - pg)
                fire_gathers(1 - pg)

            @pl.when(u < UNITS_W - 2)
            def _():
                stage_idx(u + 2, pg)

            for hl in range(8):
                hp = hl & 1
                if hl >= 2:
                    drain_out(hp)
                else:
                    @pl.when(u > 0)
                    def _():
                        drain_out(hp)

                def do_block(j, inner):
                    row_idx = pg * 1024 + hl * 128 + j * 16 + iota
                    colb = j * 16 + iota
                    cs = [(iota + k) & 31 for k in range(4)]
                    sss = [jnp.zeros((16,), jnp.float32) for _ in range(4)]
                    for _t in range(8):
                        for k in range(4):
                            g = plsc.load_gather(grows_v, [row_idx, cs[k]])
                            sss[k] = sss[k] + g * g
                            cs[k] = (cs[k] + 4) & 31
                    ss = (sss[0] + sss[1]) + (sss[2] + sss[3])
                    y = _rsqrt_newton(jnp.maximum(ss, 1e-24))
                    cs = [(iota + k) & 31 for k in range(4)]
                    for _t in range(8):
                        for k in range(4):
                            g = plsc.load_gather(grows_v, [row_idx, cs[k]])
                            plsc.store_scatter(
                                outb_v, [hp * 32 + cs[k], colb], g * y
                            )
                            cs[k] = (cs[k] + 4) & 31
                    return inner

                lax.fori_loop(0, 8, do_block, 0)

                h = h8 * 8 + hl
                for g in range(4):
                    pltpu.async_copy(
                        outb_v.at[pl.ds(hp * 32 + g * 8, 8), :],
                        out_hbm.at[h, g, b1],
                        osems[hp],
                    )
            return carry

        lax.fori_loop(0, UNITS_W, do_unit, 0)
        drain_out(0)
        drain_out(1)

    return sck


def kernel(input, W):
    a2 = (
        jnp.transpose(input, (1, 0))
        .astype(jnp.int32)
        .reshape(25, 8, 32, 128)
        .transpose(0, 2, 1, 3)
        .reshape(6400, 128)
    )
    o5 = _make_sc_call()(W, a2)
    return jnp.transpose(o5, (0, 2, 4, 1, 3)).reshape(H, B, D)
